# baseline (speedup 1.0000x reference)
import jax
import jax.numpy as jnp
from jax import lax
from jax.experimental import pallas as pl
from jax.experimental.pallas import tpu as pltpu

D_MODEL = 1024
NUM_EXPERTS = 8
SEQ = 2048
TOKEN_TILE = 256
NUM_TILES = SEQ // TOKEN_TILE


def _moe_body(x_ref, xbf_ref, shared_W_ref, shared_b_ref, router_W_ref,
              router_b_ref, expert_W_ref, expert_b_ref, out_ref,
              acc_ref, coeff_ref):
    e = pl.program_id(0)
    t = pl.program_id(1)
    tsl = pl.ds(t * TOKEN_TILE, TOKEN_TILE)

    @pl.when(e == 0)
    def _():
        xb = x_ref[tsl, :]
        scores = lax.dot_general(xb, router_W_ref[...],
                                 (((1,), (1,)), ((), ())),
                                 preferred_element_type=jnp.float32)
        scores = scores + router_b_ref[...]
        eidx = lax.broadcasted_iota(jnp.int32, scores.shape, 1)
        m0 = jnp.max(scores, axis=-1, keepdims=True)
        a0 = jnp.min(jnp.where(scores == m0, eidx, NUM_EXPERTS), axis=-1,
                     keepdims=True)
        masked = jnp.where(eidx == a0, -jnp.inf, scores)
        m1 = jnp.max(masked, axis=-1, keepdims=True)
        a1 = jnp.min(jnp.where(masked == m1, eidx, NUM_EXPERTS), axis=-1,
                     keepdims=True)
        z = jnp.exp(m1 - m0)
        w0 = 1.0 / (1.0 + z)
        w1 = z * w0
        coeff_ref[tsl, :] = (jnp.where(eidx == a0, w0, 0.0)
                             + jnp.where(eidx == a1, w1, 0.0))
        so = lax.dot_general(xbf_ref[tsl, :], shared_W_ref[...],
                             (((1,), (1,)), ((), ())),
                             preferred_element_type=jnp.float32)
        acc_ref[tsl, :] = so + shared_b_ref[...]

    xb16 = xbf_ref[tsl, :]
    we = expert_W_ref[0]
    eo = lax.dot_general(xb16, we, (((1,), (1,)), ((), ())),
                         preferred_element_type=jnp.float32)
    call = coeff_ref[tsl, :]
    lane = lax.broadcasted_iota(jnp.int32, call.shape, 1)
    coeff = jnp.sum(jnp.where(lane == e, call, 0.0), axis=1, keepdims=True)
    acc_ref[tsl, :] += coeff * (eo + expert_b_ref[0])

    @pl.when(e == NUM_EXPERTS - 1)
    def _():
        out_ref[...] = acc_ref[tsl, :]


@jax.jit
def kernel(x, shared_W, shared_b, router_W, router_b, expert_W, expert_b):
    B, S, D = x.shape
    x2 = x.reshape(S, D)
    xbf = x2.astype(jnp.bfloat16)

    grid = (NUM_EXPERTS, NUM_TILES)
    out = pl.pallas_call(
        _moe_body,
        grid=grid,
        in_specs=[
            pl.BlockSpec((S, D), lambda e, t: (0, 0)),
            pl.BlockSpec((S, D), lambda e, t: (0, 0)),
            pl.BlockSpec((D, D), lambda e, t: (0, 0)),
            pl.BlockSpec((1, D), lambda e, t: (0, 0)),
            pl.BlockSpec((NUM_EXPERTS, D), lambda e, t: (0, 0)),
            pl.BlockSpec((1, NUM_EXPERTS), lambda e, t: (0, 0)),
            pl.BlockSpec((1, D, D), lambda e, t: (e, 0, 0)),
            pl.BlockSpec((1, 1, D), lambda e, t: (e, 0, 0)),
        ],
        out_specs=pl.BlockSpec((TOKEN_TILE, D), lambda e, t: (t, 0)),
        out_shape=jax.ShapeDtypeStruct((S, D), jnp.float32),
        scratch_shapes=[pltpu.VMEM((SEQ, D), jnp.float32),
                        pltpu.VMEM((SEQ, NUM_EXPERTS), jnp.float32)],
    )(x2, xbf, shared_W.astype(jnp.bfloat16), shared_b.reshape(1, D),
      router_W, router_b.reshape(1, NUM_EXPERTS),
      expert_W.astype(jnp.bfloat16),
      expert_b.reshape(NUM_EXPERTS, 1, D))
    return out.reshape(B, S, D)


# speedup vs baseline: 1.5031x; 1.5031x over previous
import jax
import jax.numpy as jnp
from jax import lax
from jax.experimental import pallas as pl
from jax.experimental.pallas import tpu as pltpu

D_MODEL = 1024
NUM_EXPERTS = 8
SEQ = 2048
TOKEN_TILE = 256
NUM_TILES = SEQ // TOKEN_TILE


def _moe_body(x_ref, shared_W_ref, shared_b_ref, router_W_ref,
              router_b_ref, expert_W_ref, expert_b_ref, out_ref):
    xb = x_ref[...]

    scores = lax.dot_general(xb, router_W_ref[...],
                             (((1,), (1,)), ((), ())),
                             preferred_element_type=jnp.float32)
    scores = scores + router_b_ref[...]
    eidx = lax.broadcasted_iota(jnp.int32, scores.shape, 1)
    m0 = jnp.max(scores, axis=-1, keepdims=True)
    a0 = jnp.min(jnp.where(scores == m0, eidx, NUM_EXPERTS), axis=-1,
                 keepdims=True)
    masked = jnp.where(eidx == a0, -jnp.inf, scores)
    m1 = jnp.max(masked, axis=-1, keepdims=True)
    a1 = jnp.min(jnp.where(masked == m1, eidx, NUM_EXPERTS), axis=-1,
                 keepdims=True)
    z = jnp.exp(m1 - m0)
    w0 = 1.0 / (1.0 + z)
    w1 = z * w0
    coeff = jnp.where(eidx == a0, w0, 0.0) + jnp.where(eidx == a1, w1, 0.0)

    xb16 = xb.astype(jnp.bfloat16)
    acc = lax.dot_general(xb16, shared_W_ref[...], (((1,), (1,)), ((), ())),
                          preferred_element_type=jnp.float32)
    acc = acc + shared_b_ref[...]
    for e in range(NUM_EXPERTS):
        eo = lax.dot_general(xb16, expert_W_ref[e], (((1,), (1,)), ((), ())),
                             preferred_element_type=jnp.float32)
        acc = acc + coeff[:, e:e + 1] * (eo + expert_b_ref[e])
    out_ref[...] = acc


@jax.jit
def kernel(x, shared_W, shared_b, router_W, router_b, expert_W, expert_b):
    B, S, D = x.shape
    x2 = x.reshape(S, D)

    out = pl.pallas_call(
        _moe_body,
        grid=(NUM_TILES,),
        in_specs=[
            pl.BlockSpec((TOKEN_TILE, D), lambda t: (t, 0)),
            pl.BlockSpec((D, D), lambda t: (0, 0)),
            pl.BlockSpec((1, D), lambda t: (0, 0)),
            pl.BlockSpec((NUM_EXPERTS, D), lambda t: (0, 0)),
            pl.BlockSpec((1, NUM_EXPERTS), lambda t: (0, 0)),
            pl.BlockSpec((NUM_EXPERTS, D, D), lambda t: (0, 0, 0)),
            pl.BlockSpec((NUM_EXPERTS, 1, D), lambda t: (0, 0, 0)),
        ],
        out_specs=pl.BlockSpec((TOKEN_TILE, D), lambda t: (t, 0)),
        out_shape=jax.ShapeDtypeStruct((S, D), jnp.float32),
    )(x2, shared_W.astype(jnp.bfloat16), shared_b.reshape(1, D),
      router_W, router_b.reshape(1, NUM_EXPERTS),
      expert_W.astype(jnp.bfloat16),
      expert_b.reshape(NUM_EXPERTS, 1, D))
    return out.reshape(B, S, D)


# speedup vs baseline: 1.8782x; 1.2496x over previous
import jax
import jax.numpy as jnp
from jax import lax
from jax.experimental import pallas as pl
from jax.experimental.pallas import tpu as pltpu

D_MODEL = 1024
NUM_EXPERTS = 8
SEQ = 2048
TOKEN_TILE = 256
NUM_TILES = SEQ // TOKEN_TILE


def _moe_body(x_ref, shared_W_ref, shared_b_ref, router_W_ref,
              router_b_ref, expert_W_ref, expert_b_ref, out_ref):
    xb = x_ref[...]

    scores = lax.dot_general(xb, router_W_ref[...],
                             (((1,), (1,)), ((), ())),
                             preferred_element_type=jnp.float32)
    scores = scores + router_b_ref[...]
    eidx = lax.broadcasted_iota(jnp.int32, scores.shape, 1)
    m0 = jnp.max(scores, axis=-1, keepdims=True)
    a0 = jnp.min(jnp.where(scores == m0, eidx, NUM_EXPERTS), axis=-1,
                 keepdims=True)
    masked = jnp.where(eidx == a0, -jnp.inf, scores)
    m1 = jnp.max(masked, axis=-1, keepdims=True)
    a1 = jnp.min(jnp.where(masked == m1, eidx, NUM_EXPERTS), axis=-1,
                 keepdims=True)
    z = jnp.exp(m1 - m0)
    w0 = 1.0 / (1.0 + z)
    w1 = z * w0
    coeff = jnp.where(eidx == a0, w0, 0.0) + jnp.where(eidx == a1, w1, 0.0)

    acc = lax.dot_general(xb, shared_W_ref[...], (((1,), (1,)), ((), ())),
                          preferred_element_type=jnp.float32)
    acc = acc + shared_b_ref[...]
    for e in range(NUM_EXPERTS):
        eo = lax.dot_general(xb, expert_W_ref[e], (((1,), (1,)), ((), ())),
                             preferred_element_type=jnp.float32)
        acc = acc + coeff[:, e:e + 1] * (eo + expert_b_ref[e])
    out_ref[...] = acc


@jax.jit
def kernel(x, shared_W, shared_b, router_W, router_b, expert_W, expert_b):
    B, S, D = x.shape
    x2 = x.reshape(S, D)

    out = pl.pallas_call(
        _moe_body,
        grid=(NUM_TILES,),
        in_specs=[
            pl.BlockSpec((TOKEN_TILE, D), lambda t: (t, 0)),
            pl.BlockSpec((D, D), lambda t: (0, 0)),
            pl.BlockSpec((1, D), lambda t: (0, 0)),
            pl.BlockSpec((NUM_EXPERTS, D), lambda t: (0, 0)),
            pl.BlockSpec((1, NUM_EXPERTS), lambda t: (0, 0)),
            pl.BlockSpec((NUM_EXPERTS, D, D), lambda t: (0, 0, 0)),
            pl.BlockSpec((NUM_EXPERTS, 1, D), lambda t: (0, 0, 0)),
        ],
        out_specs=pl.BlockSpec((TOKEN_TILE, D), lambda t: (t, 0)),
        out_shape=jax.ShapeDtypeStruct((S, D), jnp.float32),
    )(x2, shared_W, shared_b.reshape(1, D),
      router_W, router_b.reshape(1, NUM_EXPERTS),
      expert_W, expert_b.reshape(NUM_EXPERTS, 1, D))
    return out.reshape(B, S, D)


# speedup vs baseline: 1.9250x; 1.0249x over previous
import jax
import jax.numpy as jnp
from jax import lax
from jax.experimental import pallas as pl
from jax.experimental.pallas import tpu as pltpu

D_MODEL = 1024
NUM_EXPERTS = 8
SEQ = 2048
TOKEN_TILE = 512
NUM_TILES = SEQ // TOKEN_TILE


def _moe_body(x_ref, shared_W_ref, shared_b_ref, router_W_ref,
              router_b_ref, expert_W_ref, expert_b_ref, out_ref):
    xb = x_ref[...]

    scores = lax.dot_general(xb, router_W_ref[...],
                             (((1,), (1,)), ((), ())),
                             preferred_element_type=jnp.float32)
    scores = scores + router_b_ref[...]
    eidx = lax.broadcasted_iota(jnp.int32, scores.shape, 1)
    m0 = jnp.max(scores, axis=-1, keepdims=True)
    a0 = jnp.min(jnp.where(scores == m0, eidx, NUM_EXPERTS), axis=-1,
                 keepdims=True)
    masked = jnp.where(eidx == a0, -jnp.inf, scores)
    m1 = jnp.max(masked, axis=-1, keepdims=True)
    a1 = jnp.min(jnp.where(masked == m1, eidx, NUM_EXPERTS), axis=-1,
                 keepdims=True)
    z = jnp.exp(m1 - m0)
    w0 = 1.0 / (1.0 + z)
    w1 = z * w0
    coeff = jnp.where(eidx == a0, w0, 0.0) + jnp.where(eidx == a1, w1, 0.0)

    acc = lax.dot_general(xb, shared_W_ref[...], (((1,), (1,)), ((), ())),
                          preferred_element_type=jnp.float32)
    acc = acc + shared_b_ref[...]
    for e in range(NUM_EXPERTS):
        eo = lax.dot_general(xb, expert_W_ref[e], (((1,), (1,)), ((), ())),
                             preferred_element_type=jnp.float32)
        acc = acc + coeff[:, e:e + 1] * (eo + expert_b_ref[e])
    out_ref[...] = acc


@jax.jit
def kernel(x, shared_W, shared_b, router_W, router_b, expert_W, expert_b):
    B, S, D = x.shape
    x2 = x.reshape(S, D)

    out = pl.pallas_call(
        _moe_body,
        grid=(NUM_TILES,),
        in_specs=[
            pl.BlockSpec((TOKEN_TILE, D), lambda t: (t, 0)),
            pl.BlockSpec((D, D), lambda t: (0, 0)),
            pl.BlockSpec((1, D), lambda t: (0, 0)),
            pl.BlockSpec((NUM_EXPERTS, D), lambda t: (0, 0)),
            pl.BlockSpec((1, NUM_EXPERTS), lambda t: (0, 0)),
            pl.BlockSpec((NUM_EXPERTS, D, D), lambda t: (0, 0, 0)),
            pl.BlockSpec((NUM_EXPERTS, 1, D), lambda t: (0, 0, 0)),
        ],
        out_specs=pl.BlockSpec((TOKEN_TILE, D), lambda t: (t, 0)),
        out_shape=jax.ShapeDtypeStruct((S, D), jnp.float32),
    )(x2, shared_W, shared_b.reshape(1, D),
      router_W, router_b.reshape(1, NUM_EXPERTS),
      expert_W, expert_b.reshape(NUM_EXPERTS, 1, D))
    return out.reshape(B, S, D)


# speedup vs baseline: 2.0401x; 1.0598x over previous
import jax
import jax.numpy as jnp
from jax import lax
from jax.experimental import pallas as pl
from jax.experimental.pallas import tpu as pltpu

D_MODEL = 1024
NUM_EXPERTS = 8
SEQ = 2048


def _moe_body(x_ref, shared_W_ref, shared_b_ref, router_W_ref,
              router_b_ref, expert_W_ref, expert_b_ref, out_ref, coeff_ref):
    u = pl.program_id(0)

    @pl.when(u == 0)
    def _():
        xb = x_ref[...]
        scores = lax.dot_general(xb, router_W_ref[...],
                                 (((1,), (1,)), ((), ())),
                                 preferred_element_type=jnp.float32)
        scores = scores + router_b_ref[...]
        eidx = lax.broadcasted_iota(jnp.int32, scores.shape, 1)
        m0 = jnp.max(scores, axis=-1, keepdims=True)
        a0 = jnp.min(jnp.where(scores == m0, eidx, NUM_EXPERTS), axis=-1,
                     keepdims=True)
        masked = jnp.where(eidx == a0, -jnp.inf, scores)
        m1 = jnp.max(masked, axis=-1, keepdims=True)
        a1 = jnp.min(jnp.where(masked == m1, eidx, NUM_EXPERTS), axis=-1,
                     keepdims=True)
        z = jnp.exp(m1 - m0)
        w0 = 1.0 / (1.0 + z)
        w1 = z * w0
        coeff_ref[...] = (jnp.where(eidx == a0, w0, 0.0)
                          + jnp.where(eidx == a1, w1, 0.0))
        so = lax.dot_general(xb, shared_W_ref[...], (((1,), (1,)), ((), ())),
                             preferred_element_type=jnp.float32)
        out_ref[...] = so + shared_b_ref[...]

    @pl.when(u > 0)
    def _():
        e = u - 1
        xb = x_ref[...]
        eo = lax.dot_general(xb, expert_W_ref[0], (((1,), (1,)), ((), ())),
                             preferred_element_type=jnp.float32)
        call = coeff_ref[...]
        lane = lax.broadcasted_iota(jnp.int32, call.shape, 1)
        coeff = jnp.sum(jnp.where(lane == e, call, 0.0), axis=1,
                        keepdims=True)
        out_ref[...] += coeff * (eo + expert_b_ref[0])


@jax.jit
def kernel(x, shared_W, shared_b, router_W, router_b, expert_W, expert_b):
    B, S, D = x.shape
    x2 = x.reshape(S, D)

    def _w_idx(u):
        e = jnp.maximum(u - 1, 0)
        return (e, 0, 0)

    out = pl.pallas_call(
        _moe_body,
        grid=(NUM_EXPERTS + 1,),
        in_specs=[
            pl.BlockSpec((S, D), lambda u: (0, 0)),
            pl.BlockSpec((D, D), lambda u: (0, 0)),
            pl.BlockSpec((1, D), lambda u: (0, 0)),
            pl.BlockSpec((NUM_EXPERTS, D), lambda u: (0, 0)),
            pl.BlockSpec((1, NUM_EXPERTS), lambda u: (0, 0)),
            pl.BlockSpec((1, D, D), _w_idx),
            pl.BlockSpec((1, 1, D), _w_idx),
        ],
        out_specs=pl.BlockSpec((S, D), lambda u: (0, 0)),
        out_shape=jax.ShapeDtypeStruct((S, D), jnp.float32),
        scratch_shapes=[pltpu.VMEM((S, NUM_EXPERTS), jnp.float32)],
    )(x2, shared_W, shared_b.reshape(1, D),
      router_W, router_b.reshape(1, NUM_EXPERTS),
      expert_W, expert_b.reshape(NUM_EXPERTS, 1, D))
    return out.reshape(B, S, D)
